# baseline (speedup 1.0000x reference)
import functools

import jax
import jax.numpy as jnp
from jax.experimental import pallas as pl
from jax.experimental.pallas import tpu as pltpu

CT = 512
NORM = 4


def _crf_fwd(len_ref, em_ref, t_ref, h_ref, l_ref, out_ref,
             eem_ref, a_ref, acc_ref, z_ref):
    i = pl.program_id(0)
    nchunks = pl.num_programs(0)

    E = jnp.exp(t_ref[...])
    lastE = jnp.exp(l_ref[...])
    lengths = len_ref[...]

    eem_ref[...] = jnp.exp(em_ref[...])

    def steps(a, acc, z, base, ks):
        for k in ks:
            t = base + k
            d = eem_ref[t]
            q = jnp.sum(a[:, :, None] * E[None, :, :], axis=1) * d
            zs = jnp.sum(q * lastE, axis=1, keepdims=True)
            zc = acc + jnp.log(zs)
            gt = i * CT + t
            z = jnp.where(lengths == gt + 1, zc, z)
            a = q
        s = jnp.sum(a, axis=1, keepdims=True)
        acc = acc + jnp.log(s)
        a = a / s
        return a, acc, z

    def group_body(g, carry):
        a, acc, z = carry
        return steps(a, acc, z, g * NORM, list(range(NORM)))

    @pl.when(i == 0)
    def _first_chunk():
        eh = jnp.exp(h_ref[...])
        a0 = eh * eem_ref[0]
        acc0 = jnp.zeros_like(acc_ref)
        z0 = jnp.log(jnp.sum(a0 * lastE, axis=1, keepdims=True))
        z = jnp.where(lengths == 1, z0, jnp.zeros_like(z_ref))
        a, acc, z = steps(a0, acc0, z, 0, list(range(1, NORM)))
        a, acc, z = jax.lax.fori_loop(1, CT // NORM, group_body, (a, acc, z))
        a_ref[...], acc_ref[...], z_ref[...] = a, acc, z

    @pl.when(i > 0)
    def _rest_chunks():
        carry = (a_ref[...], acc_ref[...], z_ref[...])
        a, acc, z = jax.lax.fori_loop(0, CT // NORM, group_body, carry)
        a_ref[...], acc_ref[...], z_ref[...] = a, acc, z

    @pl.when(i == nchunks - 1)
    def _emit():
        out_ref[...] = z_ref[...]


@functools.partial(jax.jit, static_argnames=("interpret",))
def kernel(emissions, transitions, head_transitions, last_transitions,
           lengths, interpret=False):
    B, T, K = emissions.shape
    em_t = jnp.transpose(emissions, (1, 0, 2))
    lengths2 = jnp.maximum(lengths, 1).reshape(B, 1)
    head2 = head_transitions.reshape(1, K)
    last2 = last_transitions.reshape(1, K)

    nchunks = T // CT
    out = pl.pallas_call(
        _crf_fwd,
        grid=(nchunks,),
        in_specs=[
            pl.BlockSpec((B, 1), lambda i: (0, 0)),
            pl.BlockSpec((CT, B, K), lambda i: (i, 0, 0)),
            pl.BlockSpec((K, K), lambda i: (0, 0)),
            pl.BlockSpec((1, K), lambda i: (0, 0)),
            pl.BlockSpec((1, K), lambda i: (0, 0)),
        ],
        out_specs=pl.BlockSpec((B, 1), lambda i: (0, 0)),
        out_shape=jax.ShapeDtypeStruct((B, 1), jnp.float32),
        scratch_shapes=[
            pltpu.VMEM((CT, B, K), jnp.float32),
            pltpu.VMEM((B, K), jnp.float32),
            pltpu.VMEM((B, 1), jnp.float32),
            pltpu.VMEM((B, 1), jnp.float32),
        ],
        compiler_params=pltpu.CompilerParams(
            dimension_semantics=("arbitrary",),
        ),
        interpret=interpret,
    )(lengths2, em_t, transitions, head2, last2)
    return out.reshape(B)


# speedup vs baseline: 1.8297x; 1.8297x over previous
import functools

import jax
import jax.numpy as jnp
from jax.experimental import pallas as pl
from jax.experimental.pallas import tpu as pltpu

CT = 512
NORM = 8


def _crf_fwd(len_ref, em_ref, t_ref, h_ref, l_ref, out_ref,
             eem_ref, a_ref, acc_ref, z_ref):
    i = pl.program_id(0)
    nchunks = pl.num_programs(0)

    E = jnp.exp(t_ref[...]).astype(jnp.bfloat16)
    lastE = jnp.exp(l_ref[...])
    lengths = len_ref[...]

    eem_ref[...] = jnp.exp(em_ref[...])

    def steps(a, acc, z, base, ks):
        for k in ks:
            t = base + k
            d = eem_ref[t]
            q = jnp.dot(a.astype(jnp.bfloat16), E,
                        preferred_element_type=jnp.float32) * d
            zs = jnp.sum(q * lastE, axis=1, keepdims=True)
            zc = acc + jnp.log(zs)
            gt = i * CT + t
            z = jnp.where(lengths == gt + 1, zc, z)
            a = q
        s = jnp.sum(a, axis=1, keepdims=True)
        acc = acc + jnp.log(s)
        a = a / s
        return a, acc, z

    def group_body(g, carry):
        a, acc, z = carry
        return steps(a, acc, z, g * NORM, list(range(NORM)))

    @pl.when(i == 0)
    def _first_chunk():
        eh = jnp.exp(h_ref[...])
        a0 = eh * eem_ref[0]
        acc0 = jnp.zeros_like(acc_ref)
        z0 = jnp.log(jnp.sum(a0 * lastE, axis=1, keepdims=True))
        z = jnp.where(lengths == 1, z0, jnp.zeros_like(z_ref))
        a, acc, z = steps(a0, acc0, z, 0, list(range(1, NORM)))
        a, acc, z = jax.lax.fori_loop(1, CT // NORM, group_body, (a, acc, z))
        a_ref[...], acc_ref[...], z_ref[...] = a, acc, z

    @pl.when(i > 0)
    def _rest_chunks():
        carry = (a_ref[...], acc_ref[...], z_ref[...])
        a, acc, z = jax.lax.fori_loop(0, CT // NORM, group_body, carry)
        a_ref[...], acc_ref[...], z_ref[...] = a, acc, z

    @pl.when(i == nchunks - 1)
    def _emit():
        out_ref[...] = z_ref[...]


@functools.partial(jax.jit, static_argnames=("interpret",))
def kernel(emissions, transitions, head_transitions, last_transitions,
           lengths, interpret=False):
    B, T, K = emissions.shape
    em_t = jnp.transpose(emissions, (1, 0, 2))
    lengths2 = jnp.maximum(lengths, 1).reshape(B, 1)
    head2 = head_transitions.reshape(1, K)
    last2 = last_transitions.reshape(1, K)

    nchunks = T // CT
    out = pl.pallas_call(
        _crf_fwd,
        grid=(nchunks,),
        in_specs=[
            pl.BlockSpec((B, 1), lambda i: (0, 0)),
            pl.BlockSpec((CT, B, K), lambda i: (i, 0, 0)),
            pl.BlockSpec((K, K), lambda i: (0, 0)),
            pl.BlockSpec((1, K), lambda i: (0, 0)),
            pl.BlockSpec((1, K), lambda i: (0, 0)),
        ],
        out_specs=pl.BlockSpec((B, 1), lambda i: (0, 0)),
        out_shape=jax.ShapeDtypeStruct((B, 1), jnp.float32),
        scratch_shapes=[
            pltpu.VMEM((CT, B, K), jnp.float32),
            pltpu.VMEM((B, K), jnp.float32),
            pltpu.VMEM((B, 1), jnp.float32),
            pltpu.VMEM((B, 1), jnp.float32),
        ],
        compiler_params=pltpu.CompilerParams(
            dimension_semantics=("arbitrary",),
        ),
        interpret=interpret,
    )(lengths2, em_t, transitions, head2, last2)
    return out.reshape(B)


# speedup vs baseline: 7.4590x; 4.0767x over previous
import functools

import jax
import jax.numpy as jnp
from jax.experimental import pallas as pl
from jax.experimental.pallas import tpu as pltpu

C = 16
W = 64
UB = 64
NORM = 8


def _crf_fwd(lr_ref, em_ref, t_ref, h_ref, l_ref, warm_ref, out_ref,
             a_ref, acc_ref, z_ref):
    g = pl.program_id(0)
    ng = pl.num_programs(0)
    R, K = a_ref.shape
    B = R // C
    S = ng * UB

    E = jnp.exp(t_ref[...]).astype(jnp.bfloat16)
    lastE = jnp.exp(l_ref[...])
    lr = lr_ref[...]

    def mm(a):
        return jnp.dot(a.astype(jnp.bfloat16), E,
                       preferred_element_type=jnp.float32)

    def warm_group(grp, a):
        for k in range(NORM):
            a = mm(a) * jnp.exp(warm_ref[grp * NORM + k])
        return a * (1.0 / jnp.sum(a, axis=1, keepdims=True))

    def main_group(a, acc, z, base, ks, init0):
        for k in ks:
            u = base + k
            d = jnp.exp(em_ref[u])
            q = mm(a) * d
            if init0 and k == 0:
                row0 = jax.lax.broadcasted_iota(jnp.int32, (R, 1), 0) < B
                q = jnp.where(row0, jnp.exp(h_ref[...]) * d, q)
                acc = jnp.zeros_like(acc)
                z = jnp.zeros_like(z)
            zs = jnp.sum(q * lastE, axis=1, keepdims=True)
            zc = acc + jnp.log(zs)
            z = jnp.where(lr == g * UB + u + 1, zc, z)
            a = q
        s = jnp.sum(a, axis=1, keepdims=True)
        acc = acc + jnp.log(s)
        a = a * (1.0 / s)
        return a, acc, z

    def group_body(grp, carry):
        a, acc, z = carry
        return main_group(a, acc, z, grp * NORM, list(range(NORM)), False)

    @pl.when(g == 0)
    def _first_block():
        a = jnp.full((R, K), 1.0 / K, jnp.float32)
        a = jax.lax.fori_loop(0, W // NORM, warm_group, a)
        acc = jnp.zeros_like(acc_ref)
        z = jnp.zeros_like(z_ref)
        a, acc, z = main_group(a, acc, z, 0, list(range(NORM)), True)
        a, acc, z = jax.lax.fori_loop(1, UB // NORM, group_body, (a, acc, z))
        a_ref[...], acc_ref[...], z_ref[...] = a, acc, z

    @pl.when(g > 0)
    def _rest_blocks():
        carry = (a_ref[...], acc_ref[...], z_ref[...])
        a, acc, z = jax.lax.fori_loop(0, UB // NORM, group_body, carry)
        a_ref[...], acc_ref[...], z_ref[...] = a, acc, z

    @pl.when(g == ng - 1)
    def _emit():
        sig = acc_ref[...].reshape(C, B)
        zl = z_ref[...].reshape(C, B)
        m = ((lr >= 1) & (lr <= S)).astype(jnp.float32).reshape(C, B)
        tri = (jax.lax.broadcasted_iota(jnp.int32, (C, C), 1)
               < jax.lax.broadcasted_iota(jnp.int32, (C, C), 0)
               ).astype(jnp.float32)
        excl = jnp.dot(tri, sig, preferred_element_type=jnp.float32)
        out_ref[...] = jnp.sum(m * (excl + zl), axis=0, keepdims=True)


@functools.partial(jax.jit, static_argnames=("interpret",))
def kernel(emissions, transitions, head_transitions, last_transitions,
           lengths, interpret=False):
    B, T, K = emissions.shape
    S = T // C
    R = C * B

    em_r = emissions.reshape(B, C, S, K).transpose(2, 1, 0, 3).reshape(S, R, K)
    em_t = jnp.transpose(emissions, (1, 0, 2))
    widx = (jnp.arange(C)[:, None] * S - W + jnp.arange(W)[None, :]) % T
    warm = em_t[widx].transpose(1, 0, 2, 3).reshape(W, R, K)

    lengths1 = jnp.maximum(lengths, 1)
    len_rel = (jnp.tile(lengths1, C)
               - jnp.repeat(jnp.arange(C, dtype=lengths.dtype) * S, B)
               ).reshape(R, 1).astype(jnp.int32)

    out = pl.pallas_call(
        _crf_fwd,
        grid=(S // UB,),
        in_specs=[
            pl.BlockSpec((R, 1), lambda g: (0, 0)),
            pl.BlockSpec((UB, R, K), lambda g: (g, 0, 0)),
            pl.BlockSpec((K, K), lambda g: (0, 0)),
            pl.BlockSpec((1, K), lambda g: (0, 0)),
            pl.BlockSpec((1, K), lambda g: (0, 0)),
            pl.BlockSpec((W, R, K), lambda g: (0, 0, 0)),
        ],
        out_specs=pl.BlockSpec((1, B), lambda g: (0, 0)),
        out_shape=jax.ShapeDtypeStruct((1, B), jnp.float32),
        scratch_shapes=[
            pltpu.VMEM((R, K), jnp.float32),
            pltpu.VMEM((R, 1), jnp.float32),
            pltpu.VMEM((R, 1), jnp.float32),
        ],
        compiler_params=pltpu.CompilerParams(
            dimension_semantics=("arbitrary",),
        ),
        interpret=interpret,
    )(len_rel, em_r, transitions, head_transitions.reshape(1, K),
      last_transitions.reshape(1, K), warm)
    return out.reshape(B)


# speedup vs baseline: 7.4678x; 1.0012x over previous
import functools

import jax
import jax.numpy as jnp
from jax.experimental import pallas as pl
from jax.experimental.pallas import tpu as pltpu

C = 16
W = 64
UB = 64
NORM = 8


def _crf_fwd(lr_ref, em_ref, t_ref, h_ref, l_ref, warm_ref, out_ref,
             a_ref, acc_ref, z_ref):
    g = pl.program_id(0)
    ng = pl.num_programs(0)
    R, K = a_ref.shape
    B = R // C
    S = ng * UB

    E = jnp.exp(t_ref[...]).astype(jnp.bfloat16)
    lastE = jnp.exp(l_ref[...])
    lr = lr_ref[...]

    def mm(a):
        return jnp.dot(a.astype(jnp.bfloat16), E,
                       preferred_element_type=jnp.float32)

    def warm_group(grp, a):
        for k in range(NORM):
            a = mm(a) * jnp.exp(warm_ref[grp * NORM + k])
        return a * (1.0 / jnp.sum(a, axis=1, keepdims=True))

    def main_group(a, acc, z, base, ks, init0):
        for k in ks:
            u = base + k
            d = jnp.exp(em_ref[u])
            q = mm(a) * d
            if init0 and k == 0:
                row0 = jax.lax.broadcasted_iota(jnp.int32, (R, 1), 0) < B
                q = jnp.where(row0, jnp.exp(h_ref[...]) * d, q)
                acc = jnp.zeros_like(acc)
                z = jnp.zeros_like(z)
            zs = jnp.sum(q * lastE, axis=1, keepdims=True)
            zc = acc + jnp.log(zs)
            z = jnp.where(lr == g * UB + u + 1, zc, z)
            a = q
        s = jnp.sum(a, axis=1, keepdims=True)
        acc = acc + jnp.log(s)
        a = a * (1.0 / s)
        return a, acc, z

    def group_body(grp, carry):
        a, acc, z = carry
        return main_group(a, acc, z, grp * NORM, list(range(NORM)), False)

    @pl.when(g == 0)
    def _first_block():
        a = jnp.full((R, K), 1.0 / K, jnp.float32)
        a = jax.lax.fori_loop(0, W // NORM, warm_group, a)
        acc = jnp.zeros_like(acc_ref)
        z = jnp.zeros_like(z_ref)
        a, acc, z = main_group(a, acc, z, 0, list(range(NORM)), True)
        a, acc, z = jax.lax.fori_loop(1, UB // NORM, group_body, (a, acc, z))
        a_ref[...], acc_ref[...], z_ref[...] = a, acc, z

    @pl.when(g > 0)
    def _rest_blocks():
        carry = (a_ref[...], acc_ref[...], z_ref[...])
        a, acc, z = jax.lax.fori_loop(0, UB // NORM, group_body, carry)
        a_ref[...], acc_ref[...], z_ref[...] = a, acc, z

    @pl.when(g == ng - 1)
    def _emit():
        sig = acc_ref[...].reshape(C, B)
        zl = z_ref[...].reshape(C, B)
        m = ((lr >= 1) & (lr <= S)).astype(jnp.float32).reshape(C, B)
        inc = sig
        k = 1
        while k < C:
            inc = inc + jnp.concatenate(
                [jnp.zeros((k, B), jnp.float32), inc[:-k]], axis=0)
            k *= 2
        excl = inc - sig
        out_ref[...] = jnp.sum(m * (excl + zl), axis=0, keepdims=True)


@functools.partial(jax.jit, static_argnames=("interpret",))
def kernel(emissions, transitions, head_transitions, last_transitions,
           lengths, interpret=False):
    B, T, K = emissions.shape
    S = T // C
    R = C * B

    em_r = emissions.reshape(B, C, S, K).transpose(2, 1, 0, 3).reshape(S, R, K)
    em_t = jnp.transpose(emissions, (1, 0, 2))
    widx = (jnp.arange(C)[:, None] * S - W + jnp.arange(W)[None, :]) % T
    warm = em_t[widx].transpose(1, 0, 2, 3).reshape(W, R, K)

    lengths1 = jnp.maximum(lengths, 1)
    len_rel = (jnp.tile(lengths1, C)
               - jnp.repeat(jnp.arange(C, dtype=lengths.dtype) * S, B)
               ).reshape(R, 1).astype(jnp.int32)

    out = pl.pallas_call(
        _crf_fwd,
        grid=(S // UB,),
        in_specs=[
            pl.BlockSpec((R, 1), lambda g: (0, 0)),
            pl.BlockSpec((UB, R, K), lambda g: (g, 0, 0)),
            pl.BlockSpec((K, K), lambda g: (0, 0)),
            pl.BlockSpec((1, K), lambda g: (0, 0)),
            pl.BlockSpec((1, K), lambda g: (0, 0)),
            pl.BlockSpec((W, R, K), lambda g: (0, 0, 0)),
        ],
        out_specs=pl.BlockSpec((1, B), lambda g: (0, 0)),
        out_shape=jax.ShapeDtypeStruct((1, B), jnp.float32),
        scratch_shapes=[
            pltpu.VMEM((R, K), jnp.float32),
            pltpu.VMEM((R, 1), jnp.float32),
            pltpu.VMEM((R, 1), jnp.float32),
        ],
        compiler_params=pltpu.CompilerParams(
            dimension_semantics=("arbitrary",),
        ),
        interpret=interpret,
    )(len_rel, em_r, transitions, head_transitions.reshape(1, K),
      last_transitions.reshape(1, K), warm)
    return out.reshape(B)


# speedup vs baseline: 9.8768x; 1.3226x over previous
import functools

import jax
import jax.numpy as jnp
from jax.experimental import pallas as pl
from jax.experimental.pallas import tpu as pltpu

C = 16
W = 64
UB = 64
NORM = 8


def _crf_fwd(lr_ref, em_ref, t_ref, h_ref, l_ref, warm_ref, out_ref,
             a_ref, acc_ref, z_ref):
    g = pl.program_id(0)
    ng = pl.num_programs(0)
    R, K = a_ref.shape
    B = R // C
    S = ng * UB

    E = jnp.exp(t_ref[...]).astype(jnp.bfloat16)
    lastE = jnp.exp(l_ref[...])
    lr = lr_ref[...]

    def mm(a):
        return jnp.dot(a.astype(jnp.bfloat16), E,
                       preferred_element_type=jnp.float32)

    def warm_group(grp, a):
        for k in range(NORM):
            a = mm(a) * jnp.exp(warm_ref[(W - UB) + grp * NORM + k])
        return a * (1.0 / jnp.sum(a, axis=1, keepdims=True))

    def main_group(a, acc, z, base, ks, init0):
        for k in ks:
            u = base + k
            d = jnp.exp(em_ref[u])
            q = mm(a) * d
            if init0 and k == 0:
                row0 = jax.lax.broadcasted_iota(jnp.int32, (R, 1), 0) < B
                q = jnp.where(row0, jnp.exp(h_ref[...]) * d, q)
                acc = jnp.zeros_like(acc)
                z = jnp.zeros_like(z)
            zs = jnp.sum(q * lastE, axis=1, keepdims=True)
            zc = acc + jnp.log(zs)
            z = jnp.where(lr == g * UB + u + 1, zc, z)
            a = q
        s = jnp.sum(a, axis=1, keepdims=True)
        acc = acc + jnp.log(s)
        a = a * (1.0 / s)
        return a, acc, z

    def group_body(grp, carry):
        a, acc, z = carry
        return main_group(a, acc, z, grp * NORM, list(range(NORM)), False)

    @pl.when(g == 0)
    def _first_block():
        a = jnp.full((R, K), 1.0 / K, jnp.float32)
        a = jax.lax.fori_loop(0, W // NORM, warm_group, a)
        a = jnp.concatenate([a[R - B:], a[:R - B]], axis=0)
        acc = jnp.zeros_like(acc_ref)
        z = jnp.zeros_like(z_ref)
        a, acc, z = main_group(a, acc, z, 0, list(range(NORM)), True)
        a, acc, z = jax.lax.fori_loop(1, UB // NORM, group_body, (a, acc, z))
        a_ref[...], acc_ref[...], z_ref[...] = a, acc, z

    @pl.when(g > 0)
    def _rest_blocks():
        carry = (a_ref[...], acc_ref[...], z_ref[...])
        a, acc, z = jax.lax.fori_loop(0, UB // NORM, group_body, carry)
        a_ref[...], acc_ref[...], z_ref[...] = a, acc, z

    @pl.when(g == ng - 1)
    def _emit():
        sig = acc_ref[...].reshape(C, B)
        zl = z_ref[...].reshape(C, B)
        m = ((lr >= 1) & (lr <= S)).astype(jnp.float32).reshape(C, B)
        inc = sig
        k = 1
        while k < C:
            inc = inc + jnp.concatenate(
                [jnp.zeros((k, B), jnp.float32), inc[:-k]], axis=0)
            k *= 2
        excl = inc - sig
        out_ref[...] = jnp.sum(m * (excl + zl), axis=0, keepdims=True)


@functools.partial(jax.jit, static_argnames=("interpret",))
def kernel(emissions, transitions, head_transitions, last_transitions,
           lengths, interpret=False):
    B, T, K = emissions.shape
    S = T // C
    R = C * B

    em_r = emissions.reshape(B, C, S, K).transpose(2, 1, 0, 3).reshape(S, R, K)

    lengths1 = jnp.maximum(lengths, 1)
    len_rel = (jnp.tile(lengths1, C)
               - jnp.repeat(jnp.arange(C, dtype=lengths.dtype) * S, B)
               ).reshape(R, 1).astype(jnp.int32)

    out = pl.pallas_call(
        _crf_fwd,
        grid=(S // UB,),
        in_specs=[
            pl.BlockSpec((R, 1), lambda g: (0, 0)),
            pl.BlockSpec((UB, R, K), lambda g: (g, 0, 0)),
            pl.BlockSpec((K, K), lambda g: (0, 0)),
            pl.BlockSpec((1, K), lambda g: (0, 0)),
            pl.BlockSpec((1, K), lambda g: (0, 0)),
            pl.BlockSpec((UB, R, K), lambda g: (S // UB - 1, 0, 0)),
        ],
        out_specs=pl.BlockSpec((1, B), lambda g: (0, 0)),
        out_shape=jax.ShapeDtypeStruct((1, B), jnp.float32),
        scratch_shapes=[
            pltpu.VMEM((R, K), jnp.float32),
            pltpu.VMEM((R, 1), jnp.float32),
            pltpu.VMEM((R, 1), jnp.float32),
        ],
        compiler_params=pltpu.CompilerParams(
            dimension_semantics=("arbitrary",),
        ),
        interpret=interpret,
    )(len_rel, em_r, transitions, head_transitions.reshape(1, K),
      last_transitions.reshape(1, K), em_r)
    return out.reshape(B)


# speedup vs baseline: 9.9882x; 1.0113x over previous
import functools

import jax
import jax.numpy as jnp
from jax.experimental import pallas as pl
from jax.experimental.pallas import tpu as pltpu

C = 16
W = 64
UB = 64
NORM = 8


def _crf_fwd(lr_ref, em_ref, t_ref, h_ref, l_ref, warm_ref, out_ref,
             eem_ref, a_ref, acc_ref, z_ref):
    g = pl.program_id(0)
    ng = pl.num_programs(0)
    R, K = a_ref.shape
    B = R // C
    S = ng * UB

    E = jnp.exp(t_ref[...]).astype(jnp.bfloat16)
    lastE = jnp.exp(l_ref[...])
    lr = lr_ref[...]

    def mm(a):
        return jnp.dot(a.astype(jnp.bfloat16), E,
                       preferred_element_type=jnp.float32)

    def relayout(ref):
        blk = ref[...].reshape(R, UB, K)
        eem_ref[...] = jnp.exp(jnp.transpose(blk, (1, 0, 2)))

    def warm_group(grp, a):
        for k in range(NORM):
            a = mm(a) * eem_ref[grp * NORM + k]
        return a * (1.0 / jnp.sum(a, axis=1, keepdims=True))

    def main_group(a, acc, z, base, ks, init0):
        for k in ks:
            u = base + k
            d = eem_ref[u]
            q = mm(a) * d
            if init0 and k == 0:
                row0 = (jax.lax.broadcasted_iota(jnp.int32, (R, 1), 0)
                        & (C - 1)) == 0
                q = jnp.where(row0, jnp.exp(h_ref[...]) * d, q)
                acc = jnp.zeros_like(acc)
                z = jnp.zeros_like(z)
            zs = jnp.sum(q * lastE, axis=1, keepdims=True)
            zc = acc + jnp.log(zs)
            z = jnp.where(lr == g * UB + u + 1, zc, z)
            a = q
        s = jnp.sum(a, axis=1, keepdims=True)
        acc = acc + jnp.log(s)
        a = a * (1.0 / s)
        return a, acc, z

    def group_body(grp, carry):
        a, acc, z = carry
        return main_group(a, acc, z, grp * NORM, list(range(NORM)), False)

    @pl.when(g == 0)
    def _first_block():
        relayout(warm_ref)
        a = jnp.full((R, K), 1.0 / K, jnp.float32)
        a = jax.lax.fori_loop(0, W // NORM, warm_group, a)
        a = jnp.concatenate([a[R - 1:], a[:R - 1]], axis=0)
        relayout(em_ref)
        acc = jnp.zeros_like(acc_ref)
        z = jnp.zeros_like(z_ref)
        a, acc, z = main_group(a, acc, z, 0, list(range(NORM)), True)
        a, acc, z = jax.lax.fori_loop(1, UB // NORM, group_body, (a, acc, z))
        a_ref[...], acc_ref[...], z_ref[...] = a, acc, z

    @pl.when(g > 0)
    def _rest_blocks():
        relayout(em_ref)
        carry = (a_ref[...], acc_ref[...], z_ref[...])
        a, acc, z = jax.lax.fori_loop(0, UB // NORM, group_body, carry)
        a_ref[...], acc_ref[...], z_ref[...] = a, acc, z

    @pl.when(g == ng - 1)
    def _emit():
        sig = acc_ref[...].reshape(B, C)
        zl = z_ref[...].reshape(B, C)
        m = ((lr >= 1) & (lr <= S)).astype(jnp.float32).reshape(B, C)
        inc = sig
        k = 1
        while k < C:
            inc = inc + jnp.concatenate(
                [jnp.zeros((B, k), jnp.float32), inc[:, :-k]], axis=1)
            k *= 2
        excl = inc - sig
        out_ref[...] = jnp.sum(m * (excl + zl), axis=1, keepdims=True)


@functools.partial(jax.jit, static_argnames=("interpret",))
def kernel(emissions, transitions, head_transitions, last_transitions,
           lengths, interpret=False):
    B, T, K = emissions.shape
    S = T // C
    R = B * C

    em4 = emissions.reshape(B, C, S, K)
    lengths1 = jnp.maximum(lengths, 1)
    len_rel = (jnp.repeat(lengths1, C)
               - jnp.tile(jnp.arange(C, dtype=lengths.dtype) * S, B)
               ).reshape(R, 1).astype(jnp.int32)

    out = pl.pallas_call(
        _crf_fwd,
        grid=(S // UB,),
        in_specs=[
            pl.BlockSpec((R, 1), lambda g: (0, 0)),
            pl.BlockSpec((B, C, UB, K), lambda g: (0, 0, g, 0)),
            pl.BlockSpec((K, K), lambda g: (0, 0)),
            pl.BlockSpec((1, K), lambda g: (0, 0)),
            pl.BlockSpec((1, K), lambda g: (0, 0)),
            pl.BlockSpec((B, C, UB, K), lambda g: (0, 0, S // UB - 1, 0)),
        ],
        out_specs=pl.BlockSpec((R // C, 1), lambda g: (0, 0)),
        out_shape=jax.ShapeDtypeStruct((B, 1), jnp.float32),
        scratch_shapes=[
            pltpu.VMEM((UB, R, K), jnp.float32),
            pltpu.VMEM((R, K), jnp.float32),
            pltpu.VMEM((R, 1), jnp.float32),
            pltpu.VMEM((R, 1), jnp.float32),
        ],
        compiler_params=pltpu.CompilerParams(
            dimension_semantics=("arbitrary",),
        ),
        interpret=interpret,
    )(len_rel, em4, transitions, head_transitions.reshape(1, K),
      last_transitions.reshape(1, K), em4)
    return out.reshape(B)


# speedup vs baseline: 10.1137x; 1.0126x over previous
import functools

import jax
import jax.numpy as jnp
from jax.experimental import pallas as pl
from jax.experimental.pallas import tpu as pltpu

C = 16
W = 64
UB = 64
NORM = 8


def _crf_fwd(lr_ref, em_ref, t_ref, h_ref, l_ref, out_ref,
             eem_ref, a_ref, acc_ref, z_ref):
    g = pl.program_id(0)
    ng = pl.num_programs(0)
    R, K = a_ref.shape
    B = R // C
    S = (ng - 1) * UB

    E = jnp.exp(t_ref[...]).astype(jnp.bfloat16)
    lastE = jnp.exp(l_ref[...])
    lr = lr_ref[...]

    def mm(a):
        return jnp.dot(a.astype(jnp.bfloat16), E,
                       preferred_element_type=jnp.float32)

    def relayout(ref):
        blk = ref[...].reshape(R, UB, K)
        eem_ref[...] = jnp.exp(jnp.transpose(blk, (1, 0, 2)))

    def warm_group(grp, a):
        for k in range(NORM):
            a = mm(a) * eem_ref[grp * NORM + k]
        return a * (1.0 / jnp.sum(a, axis=1, keepdims=True))

    def main_group(a, acc, z, base, ks, init0):
        for k in ks:
            u = base + k
            d = eem_ref[u]
            q = mm(a) * d
            if init0 and k == 0:
                row0 = (jax.lax.broadcasted_iota(jnp.int32, (R, 1), 0)
                        & (C - 1)) == 0
                q = jnp.where(row0, jnp.exp(h_ref[...]) * d, q)
                acc = jnp.zeros_like(acc)
                z = jnp.zeros_like(z)
            zs = jnp.sum(q * lastE, axis=1, keepdims=True)
            zc = acc + jnp.log(zs)
            z = jnp.where(lr == (g - 1) * UB + u + 1, zc, z)
            a = q
        s = jnp.sum(a, axis=1, keepdims=True)
        acc = acc + jnp.log(s)
        a = a * (1.0 / s)
        return a, acc, z

    def group_body(grp, carry):
        a, acc, z = carry
        return main_group(a, acc, z, grp * NORM, list(range(NORM)), False)

    @pl.when(g == 0)
    def _warm_block():
        relayout(em_ref)
        a = jnp.full((R, K), 1.0 / K, jnp.float32)
        a = jax.lax.fori_loop(0, W // NORM, warm_group, a)
        a_ref[...] = jnp.concatenate([a[R - 1:], a[:R - 1]], axis=0)

    @pl.when(g == 1)
    def _first_block():
        relayout(em_ref)
        a = a_ref[...]
        acc = jnp.zeros_like(acc_ref)
        z = jnp.zeros_like(z_ref)
        a, acc, z = main_group(a, acc, z, 0, list(range(NORM)), True)
        a, acc, z = jax.lax.fori_loop(1, UB // NORM, group_body, (a, acc, z))
        a_ref[...], acc_ref[...], z_ref[...] = a, acc, z

    @pl.when(g > 1)
    def _rest_blocks():
        relayout(em_ref)
        carry = (a_ref[...], acc_ref[...], z_ref[...])
        a, acc, z = jax.lax.fori_loop(0, UB // NORM, group_body, carry)
        a_ref[...], acc_ref[...], z_ref[...] = a, acc, z

    @pl.when(g == ng - 1)
    def _emit():
        sig = acc_ref[...].reshape(B, C)
        zl = z_ref[...].reshape(B, C)
        m = ((lr >= 1) & (lr <= S)).astype(jnp.float32).reshape(B, C)
        inc = sig
        k = 1
        while k < C:
            inc = inc + jnp.concatenate(
                [jnp.zeros((B, k), jnp.float32), inc[:, :-k]], axis=1)
            k *= 2
        excl = inc - sig
        out_ref[...] = jnp.sum(m * (excl + zl), axis=1, keepdims=True)


@functools.partial(jax.jit, static_argnames=("interpret",))
def kernel(emissions, transitions, head_transitions, last_transitions,
           lengths, interpret=False):
    B, T, K = emissions.shape
    S = T // C
    R = B * C

    em4 = emissions.reshape(B, C, S, K)
    lengths1 = jnp.maximum(lengths, 1)
    len_rel = (jnp.repeat(lengths1, C)
               - jnp.tile(jnp.arange(C, dtype=lengths.dtype) * S, B)
               ).reshape(R, 1).astype(jnp.int32)

    ngb = S // UB
    out = pl.pallas_call(
        _crf_fwd,
        grid=(ngb + 1,),
        in_specs=[
            pl.BlockSpec((R, 1), lambda g: (0, 0)),
            pl.BlockSpec((B, C, UB, K),
                         lambda g: (0, 0, (g + ngb - 1) % ngb, 0)),
            pl.BlockSpec((K, K), lambda g: (0, 0)),
            pl.BlockSpec((1, K), lambda g: (0, 0)),
            pl.BlockSpec((1, K), lambda g: (0, 0)),
        ],
        out_specs=pl.BlockSpec((R // C, 1), lambda g: (0, 0)),
        out_shape=jax.ShapeDtypeStruct((B, 1), jnp.float32),
        scratch_shapes=[
            pltpu.VMEM((UB, R, K), jnp.float32),
            pltpu.VMEM((R, K), jnp.float32),
            pltpu.VMEM((R, 1), jnp.float32),
            pltpu.VMEM((R, 1), jnp.float32),
        ],
        compiler_params=pltpu.CompilerParams(
            dimension_semantics=("arbitrary",),
        ),
        interpret=interpret,
    )(len_rel, em4, transitions, head_transitions.reshape(1, K),
      last_transitions.reshape(1, K))
    return out.reshape(B)


# speedup vs baseline: 10.8060x; 1.0684x over previous
import functools

import jax
import jax.numpy as jnp
from jax.experimental import pallas as pl
from jax.experimental.pallas import tpu as pltpu

C = 16
W = 64
UB = 64
NORM = 8


def _crf_fwd(lr_ref, em_ref, t_ref, h_ref, l_ref, out_ref,
             eem_ref, a_ref, acc_ref, z_ref):
    g = pl.program_id(0)
    ng = pl.num_programs(0)
    R, K = a_ref.shape
    B = R // C
    S = (ng - 1) * UB

    E = jnp.exp(t_ref[...]).astype(jnp.bfloat16)
    lastE = jnp.exp(l_ref[...])
    lr = lr_ref[...]

    def mm(a):
        return jnp.dot(a.astype(jnp.bfloat16), E,
                       preferred_element_type=jnp.float32)

    def relayout(base):
        for c in range(C):
            sub = em_ref[:, pl.ds(c * S + base, UB), :]
            eem_ref[:, c * B:(c + 1) * B, :] = jnp.exp(
                jnp.transpose(sub, (1, 0, 2)))

    def warm_group(grp, a):
        for k in range(NORM):
            a = mm(a) * eem_ref[grp * NORM + k]
        return a * (1.0 / jnp.sum(a, axis=1, keepdims=True))

    def main_group(a, acc, z, base, ks, init0):
        for k in ks:
            u = base + k
            d = eem_ref[u]
            q = mm(a) * d
            if init0 and k == 0:
                row0 = jax.lax.broadcasted_iota(jnp.int32, (R, 1), 0) < B
                q = jnp.where(row0, jnp.exp(h_ref[...]) * d, q)
                acc = jnp.zeros_like(acc)
                z = jnp.zeros_like(z)
            zs = jnp.sum(q * lastE, axis=1, keepdims=True)
            zc = acc + jnp.log(zs)
            z = jnp.where(lr == (g - 1) * UB + u + 1, zc, z)
            a = q
        s = jnp.sum(a, axis=1, keepdims=True)
        acc = acc + jnp.log(s)
        a = a * (1.0 / s)
        return a, acc, z

    def group_body(grp, carry):
        a, acc, z = carry
        return main_group(a, acc, z, grp * NORM, list(range(NORM)), False)

    @pl.when(g == 0)
    def _warm_block():
        relayout(S - UB)
        a = jnp.full((R, K), 1.0 / K, jnp.float32)
        a = jax.lax.fori_loop(0, W // NORM, warm_group, a)
        a_ref[...] = jnp.concatenate([a[R - B:], a[:R - B]], axis=0)

    @pl.when(g == 1)
    def _first_block():
        relayout(0)
        a = a_ref[...]
        acc = jnp.zeros_like(acc_ref)
        z = jnp.zeros_like(z_ref)
        a, acc, z = main_group(a, acc, z, 0, list(range(NORM)), True)
        a, acc, z = jax.lax.fori_loop(1, UB // NORM, group_body, (a, acc, z))
        a_ref[...], acc_ref[...], z_ref[...] = a, acc, z

    @pl.when(g > 1)
    def _rest_blocks():
        relayout((g - 1) * UB)
        carry = (a_ref[...], acc_ref[...], z_ref[...])
        a, acc, z = jax.lax.fori_loop(0, UB // NORM, group_body, carry)
        a_ref[...], acc_ref[...], z_ref[...] = a, acc, z

    @pl.when(g == ng - 1)
    def _emit():
        sig = acc_ref[...].reshape(C, B)
        zl = z_ref[...].reshape(C, B)
        m = ((lr >= 1) & (lr <= S)).astype(jnp.float32).reshape(C, B)
        inc = sig
        k = 1
        while k < C:
            inc = inc + jnp.concatenate(
                [jnp.zeros((k, B), jnp.float32), inc[:-k]], axis=0)
            k *= 2
        excl = inc - sig
        out_ref[...] = jnp.sum(m * (excl + zl), axis=0, keepdims=True)


@functools.partial(jax.jit, static_argnames=("interpret",))
def kernel(emissions, transitions, head_transitions, last_transitions,
           lengths, interpret=False):
    B, T, K = emissions.shape
    S = T // C
    R = C * B

    lengths1 = jnp.maximum(lengths, 1)
    len_rel = (jnp.tile(lengths1, C)
               - jnp.repeat(jnp.arange(C, dtype=lengths.dtype) * S, B)
               ).reshape(R, 1).astype(jnp.int32)

    out = pl.pallas_call(
        _crf_fwd,
        grid=(S // UB + 1,),
        in_specs=[
            pl.BlockSpec((R, 1), lambda g: (0, 0)),
            pl.BlockSpec((B, T, K), lambda g: (0, 0, 0)),
            pl.BlockSpec((K, K), lambda g: (0, 0)),
            pl.BlockSpec((1, K), lambda g: (0, 0)),
            pl.BlockSpec((1, K), lambda g: (0, 0)),
        ],
        out_specs=pl.BlockSpec((1, B), lambda g: (0, 0)),
        out_shape=jax.ShapeDtypeStruct((1, B), jnp.float32),
        scratch_shapes=[
            pltpu.VMEM((UB, R, K), jnp.float32),
            pltpu.VMEM((R, K), jnp.float32),
            pltpu.VMEM((R, 1), jnp.float32),
            pltpu.VMEM((R, 1), jnp.float32),
        ],
        compiler_params=pltpu.CompilerParams(
            dimension_semantics=("arbitrary",),
        ),
        interpret=interpret,
    )(len_rel, emissions, transitions, head_transitions.reshape(1, K),
      last_transitions.reshape(1, K))
    return out.reshape(B)
